# baseline (speedup 1.0000x reference)
import jax
import jax.numpy as jnp
from jax.experimental import pallas as pl
from jax.experimental.pallas import tpu as pltpu

H = 512
C = 1000
CPAD = 1024
SEQ = 4096
WIN = 4
MAX_STEP = 8
BLK = 16
NEG = -1e30


def _scan_start(pos):
    return pl.multiple_of(
        jnp.minimum(jnp.maximum(pos - (WIN - 1), 0) & ~7, SEQ - BLK), 8)


def _fused(enc_ref, lys_ref,
           wsm_ref, wsmb_ref, whm_ref, vm_ref, gm_ref, rm_ref,
           wsc_ref, wscb_ref, whc_ref, vc_ref,
           lsy_ref, lgy_ref, lgyb_ref, lyy_ref, lyyb_ref,
           lss_ref, lgs_ref, lgsb_ref,
           out_ref,
           blk_vmem, row_vmem, lss_vmem, lgs_vmem,
           sem_blk, sem_row, sem_w0, sem_w1, sem_w2, sem_w3):
    f32 = jnp.float32

    def dott(a, b):
        return jax.lax.dot_general(a, b, (((1,), (1,)), ((), ())),
                                   preferred_element_type=f32)

    def blk_copy(start):
        return pltpu.make_async_copy(
            enc_ref.at[pl.ds(start, BLK), :], blk_vmem, sem_blk)

    def rec_weight_copies():
        return (
            pltpu.make_async_copy(
                lss_ref.at[pl.ds(0, 2 * H), :],
                lss_vmem.at[pl.ds(0, 2 * H), :], sem_w0),
            pltpu.make_async_copy(
                lss_ref.at[pl.ds(2 * H, 2 * H), :],
                lss_vmem.at[pl.ds(2 * H, 2 * H), :], sem_w1),
            pltpu.make_async_copy(
                lgs_ref.at[pl.ds(0, 2 * H), :],
                lgs_vmem.at[pl.ds(0, 2 * H), :], sem_w2),
            pltpu.make_async_copy(
                lgs_ref.at[pl.ds(2 * H, 2 * H), :],
                lgs_vmem.at[pl.ds(2 * H, 2 * H), :], sem_w3),
        )

    v_mono = vm_ref[...]
    v_norm_mono = gm_ref[0, 0] * jax.lax.rsqrt(jnp.sum(v_mono * v_mono))
    r_mono = rm_ref[0, 0]

    def energies(start, cur_idx, bias_mono):
        t = jnp.tanh(dott(blk_vmem[...], whm_ref[...]) + bias_mono)
        e = v_norm_mono * dott(t, v_mono) + r_mono
        p = jax.nn.sigmoid(e)
        jg = start + jax.lax.broadcasted_iota(jnp.int32, (BLK, 1), 0)
        ok = jnp.logical_and(p >= 0.5, jg >= cur_idx)
        m = jnp.min(jnp.where(ok, jg, SEQ))
        return m < SEQ, m

    def step_body(step, carry):
        s, c, cur_idx, alive = carry
        start0 = _scan_start(cur_idx)
        bias_mono = dott(s, wsm_ref[...]) + wsmb_ref[...]
        blk_copy(start0).wait()
        found0, fidx0 = energies(start0, cur_idx, bias_mono)

        def scan_cond(sc):
            st, _fidx, found = sc
            return jnp.logical_and(
                jnp.logical_and(alive, jnp.logical_not(found)),
                st + BLK < SEQ)

        def scan_body(sc):
            st, fidx, _found = sc
            start = pl.multiple_of(
                jnp.minimum(st + (BLK - 8), SEQ - BLK), 8)
            cp = blk_copy(start)
            cp.start()
            cp.wait()
            found_new, m = energies(start, cur_idx, bias_mono)
            fidx_new = jnp.where(found_new, m, fidx)
            return (start, fidx_new, found_new)

        start_f, fidx, found = jax.lax.while_loop(
            scan_cond, scan_body, (start0, fidx0, found0))
        any_fired = jnp.logical_and(found, alive)
        fired_index = jnp.where(any_fired, fidx, 0)

        refetch = jnp.logical_and(jnp.logical_not(found), start_f != 0)
        start_eff = jnp.where(refetch, 0, start_f)

        @pl.when(refetch)
        def _refetch_block0():
            cp = blk_copy(jnp.int32(0))
            cp.start()
            cp.wait()

        bias_chunk = dott(s, wsc_ref[...]) + wscb_ref[...]
        tw = jnp.tanh(dott(blk_vmem[...], whc_ref[...]) + bias_chunk)
        ec = dott(tw, vc_ref[...])
        jg = start_eff + jax.lax.broadcasted_iota(jnp.int32, (BLK, 1), 0)
        wstart = jnp.maximum(fired_index - (WIN - 1), 0)
        in_win = jnp.logical_and(jg >= wstart, jg < wstart + WIN)
        validf = jnp.logical_and(in_win, jg <= fired_index)
        en = jnp.where(in_win, ec * validf.astype(f32), NEG)
        b = jnp.exp(en - jnp.max(en))
        beta = b / jnp.sum(b)
        coef = jnp.where(validf, beta, 0.0)
        wm = jnp.where(validf, blk_vmem[...], 0.0)
        context = jnp.sum(wm * coef, axis=0, keepdims=True)

        blk_copy(_scan_start(fired_index)).start()

        h = jnp.tanh(dott(context, lgy_ref[...]) + lgyb_ref[...]
                     + dott(s, lsy_ref[...]))
        y = dott(h, lyy_ref[...]) + lyyb_ref[...]
        out_ref[pl.ds(step, 1), :] = y
        ymax = jnp.max(y)
        col = jax.lax.broadcasted_iota(jnp.int32, (1, CPAD), 1)
        tok = jnp.min(jnp.where(y == ymax, col, CPAD))

        ta = pl.multiple_of(jnp.minimum(tok & ~7, C - 8), 8)
        cpr = pltpu.make_async_copy(
            lys_ref.at[pl.ds(ta, 8), :], row_vmem, sem_row)
        cpr.start()

        @pl.when(step == 0)
        def _wait_recurrent_weights():
            for cp in rec_weight_copies():
                cp.wait()

        rec_mm = (dott(s, lss_vmem[...]) + dott(context, lgs_vmem[...])
                  + lgsb_ref[...])
        cpr.wait()
        rsel = (ta + jax.lax.broadcasted_iota(jnp.int32, (8, 1), 0)) == tok
        ys_row = jnp.sum(jnp.where(rsel, row_vmem[...], 0.0),
                         axis=0, keepdims=True)
        rec = ys_row + rec_mm
        ig = rec[:, 0:H]
        fg = rec[:, H:2 * H]
        gg = rec[:, 2 * H:3 * H]
        og = rec[:, 3 * H:4 * H]
        c_new = jax.nn.sigmoid(fg) * c + jax.nn.sigmoid(ig) * jnp.tanh(gg)
        s_new = jax.nn.sigmoid(og) * jnp.tanh(c_new)
        s = jnp.where(any_fired, s_new, s)
        c = jnp.where(any_fired, c_new, c)
        return (s, c, fired_index, any_fired)

    for cp in rec_weight_copies():
        cp.start()
    blk_copy(_scan_start(jnp.int32(0))).start()
    s0 = jnp.zeros((1, H), f32)
    c0 = jnp.zeros((1, H), f32)
    s, c, cur_idx, alive = jax.lax.fori_loop(
        0, MAX_STEP, step_body, (s0, c0, jnp.int32(0), jnp.bool_(True)))
    blk_copy(_scan_start(cur_idx)).wait()


def kernel(enc_output_, x, W_s_mono_w, W_s_mono_b, W_h_mono_w, v_mono_w,
           g_mono, r_mono, W_s_chunk_w, W_s_chunk_b, W_h_chunk_w, v_chunk_w,
           L_sy_w, L_gy_w, L_gy_b, L_yy_w, L_yy_b, L_ys_w, L_ss_w, L_gs_w,
           L_gs_b):
    del x
    f32 = jnp.float32
    enc = enc_output_.reshape(SEQ, 2 * H)
    lyy_pad = jnp.concatenate([L_yy_w, jnp.zeros((CPAD - C, H), f32)], axis=0)
    lyyb_pad = jnp.concatenate([L_yy_b, jnp.full((CPAD - C,), NEG, f32)]
                               )[None, :]

    vmem = pl.BlockSpec(memory_space=pltpu.VMEM)
    anymem = pl.BlockSpec(memory_space=pltpu.MemorySpace.HBM)

    out = pl.pallas_call(
        _fused,
        out_shape=jax.ShapeDtypeStruct((MAX_STEP, CPAD), f32),
        in_specs=([anymem, anymem] + [vmem] * 15 + [anymem, anymem]
                  + [vmem]),
        out_specs=vmem,
        scratch_shapes=[
            pltpu.VMEM((BLK, 2 * H), f32),
            pltpu.VMEM((8, 4 * H), f32),
            pltpu.VMEM((4 * H, H), f32),
            pltpu.VMEM((4 * H, 2 * H), f32),
            pltpu.SemaphoreType.DMA,
            pltpu.SemaphoreType.DMA,
            pltpu.SemaphoreType.DMA,
            pltpu.SemaphoreType.DMA,
            pltpu.SemaphoreType.DMA,
            pltpu.SemaphoreType.DMA,
        ],
        compiler_params=pltpu.CompilerParams(
            vmem_limit_bytes=100 * 1024 * 1024,
        ),
    )(enc, L_ys_w,
      W_s_mono_w, W_s_mono_b.reshape(1, 2 * H), W_h_mono_w, v_mono_w,
      g_mono.reshape(1, 1), r_mono.reshape(1, 1),
      W_s_chunk_w, W_s_chunk_b.reshape(1, 2 * H), W_h_chunk_w, v_chunk_w,
      L_sy_w, L_gy_w, L_gy_b.reshape(1, H), lyy_pad, lyyb_pad,
      L_ss_w, L_gs_w, L_gs_b.reshape(1, 4 * H))
    return out[:, :C]


# speedup vs baseline: 1.0644x; 1.0644x over previous
import jax
import jax.numpy as jnp
from jax.experimental import pallas as pl
from jax.experimental.pallas import tpu as pltpu

H = 512
C = 1000
CPAD = 1024
SEQ = 4096
WIN = 4
MAX_STEP = 8
BLK = 16
NEG = -1e30


def _scan_start(pos):
    return pl.multiple_of(
        jnp.minimum(jnp.maximum(pos - (WIN - 1), 0) & ~7, SEQ - BLK), 8)


def _fused(enc_ref, lys_ref,
           wsm_ref, wsmb_ref, whm_ref, vm_ref, gm_ref, rm_ref,
           wsc_ref, wscb_ref, whc_ref, vc_ref,
           lsy_ref, lgy_ref, lgyb_ref, lyy_ref, lyyb_ref,
           lss_ref, lgs_ref, lgsb_ref,
           out_ref,
           blk_vmem, row_vmem, sem_blk, sem_row):
    f32 = jnp.float32

    def dott(a, b):
        return jax.lax.dot_general(a, b, (((1,), (1,)), ((), ())),
                                   preferred_element_type=f32)

    def blk_copy(start):
        return pltpu.make_async_copy(
            enc_ref.at[pl.ds(start, BLK), :], blk_vmem, sem_blk)

    v_mono = vm_ref[...]
    v_norm_mono = gm_ref[0, 0] * jax.lax.rsqrt(jnp.sum(v_mono * v_mono))
    r_mono = rm_ref[0, 0]

    def energies(start, cur_idx, bias_mono):
        t = jnp.tanh(dott(blk_vmem[...], whm_ref[...]) + bias_mono)
        e = v_norm_mono * dott(t, v_mono) + r_mono
        p = jax.nn.sigmoid(e)
        jg = start + jax.lax.broadcasted_iota(jnp.int32, (BLK, 1), 0)
        ok = jnp.logical_and(p >= 0.5, jg >= cur_idx)
        m = jnp.min(jnp.where(ok, jg, SEQ))
        return m < SEQ, m

    def step_body(step, carry):
        s, c, cur_idx, alive = carry
        start0 = _scan_start(cur_idx)
        bias_mono = dott(s, wsm_ref[...]) + wsmb_ref[...]
        blk_copy(start0).wait()
        found0, fidx0 = energies(start0, cur_idx, bias_mono)

        def scan_cond(sc):
            st, _fidx, found = sc
            return jnp.logical_and(
                jnp.logical_and(alive, jnp.logical_not(found)),
                st + BLK < SEQ)

        def scan_body(sc):
            st, fidx, _found = sc
            start = pl.multiple_of(
                jnp.minimum(st + (BLK - 8), SEQ - BLK), 8)
            cp = blk_copy(start)
            cp.start()
            cp.wait()
            found_new, m = energies(start, cur_idx, bias_mono)
            fidx_new = jnp.where(found_new, m, fidx)
            return (start, fidx_new, found_new)

        start_f, fidx, found = jax.lax.while_loop(
            scan_cond, scan_body, (start0, fidx0, found0))
        any_fired = jnp.logical_and(found, alive)
        fired_index = jnp.where(any_fired, fidx, 0)

        refetch = jnp.logical_and(jnp.logical_not(found), start_f != 0)
        start_eff = jnp.where(refetch, 0, start_f)

        @pl.when(refetch)
        def _refetch_block0():
            cp = blk_copy(jnp.int32(0))
            cp.start()
            cp.wait()

        bias_chunk = dott(s, wsc_ref[...]) + wscb_ref[...]
        tw = jnp.tanh(dott(blk_vmem[...], whc_ref[...]) + bias_chunk)
        ec = dott(tw, vc_ref[...])
        jg = start_eff + jax.lax.broadcasted_iota(jnp.int32, (BLK, 1), 0)
        wstart = jnp.maximum(fired_index - (WIN - 1), 0)
        in_win = jnp.logical_and(jg >= wstart, jg < wstart + WIN)
        validf = jnp.logical_and(in_win, jg <= fired_index)
        en = jnp.where(in_win, ec * validf.astype(f32), NEG)
        b = jnp.exp(en - jnp.max(en))
        beta = b / jnp.sum(b)
        coef = jnp.where(validf, beta, 0.0)
        wm = jnp.where(validf, blk_vmem[...], 0.0)
        context = jnp.sum(wm * coef, axis=0, keepdims=True)

        blk_copy(_scan_start(fired_index)).start()

        h = jnp.tanh(dott(context, lgy_ref[...]) + lgyb_ref[...]
                     + dott(s, lsy_ref[...]))
        y = dott(h, lyy_ref[...]) + lyyb_ref[...]
        out_ref[pl.ds(step, 1), :] = y
        ymax = jnp.max(y)
        col = jax.lax.broadcasted_iota(jnp.int32, (1, CPAD), 1)
        tok = jnp.min(jnp.where(y == ymax, col, CPAD))

        ta = pl.multiple_of(jnp.minimum(tok & ~7, C - 8), 8)
        cpr = pltpu.make_async_copy(
            lys_ref.at[pl.ds(ta, 8), :], row_vmem, sem_row)
        cpr.start()
        rec_mm = (dott(s, lss_ref[...]) + dott(context, lgs_ref[...])
                  + lgsb_ref[...])
        cpr.wait()
        rsel = (ta + jax.lax.broadcasted_iota(jnp.int32, (8, 1), 0)) == tok
        ys_row = jnp.sum(jnp.where(rsel, row_vmem[...], 0.0),
                         axis=0, keepdims=True)
        rec = ys_row + rec_mm
        ig = rec[:, 0:H]
        fg = rec[:, H:2 * H]
        gg = rec[:, 2 * H:3 * H]
        og = rec[:, 3 * H:4 * H]
        c_new = jax.nn.sigmoid(fg) * c + jax.nn.sigmoid(ig) * jnp.tanh(gg)
        s_new = jax.nn.sigmoid(og) * jnp.tanh(c_new)
        s = jnp.where(any_fired, s_new, s)
        c = jnp.where(any_fired, c_new, c)
        return (s, c, fired_index, any_fired)

    blk_copy(_scan_start(jnp.int32(0))).start()
    s0 = jnp.zeros((1, H), f32)
    c0 = jnp.zeros((1, H), f32)
    s, c, cur_idx, alive = jax.lax.fori_loop(
        0, MAX_STEP, step_body, (s0, c0, jnp.int32(0), jnp.bool_(True)))
    blk_copy(_scan_start(cur_idx)).wait()


def kernel(enc_output_, x, W_s_mono_w, W_s_mono_b, W_h_mono_w, v_mono_w,
           g_mono, r_mono, W_s_chunk_w, W_s_chunk_b, W_h_chunk_w, v_chunk_w,
           L_sy_w, L_gy_w, L_gy_b, L_yy_w, L_yy_b, L_ys_w, L_ss_w, L_gs_w,
           L_gs_b):
    del x
    f32 = jnp.float32
    enc = enc_output_.reshape(SEQ, 2 * H)
    lyy_pad = jnp.concatenate([L_yy_w, jnp.zeros((CPAD - C, H), f32)], axis=0)
    lyyb_pad = jnp.concatenate([L_yy_b, jnp.full((CPAD - C,), NEG, f32)]
                               )[None, :]

    vmem = pl.BlockSpec(memory_space=pltpu.VMEM)
    anymem = pl.BlockSpec(memory_space=pltpu.MemorySpace.HBM)

    out = pl.pallas_call(
        _fused,
        out_shape=jax.ShapeDtypeStruct((MAX_STEP, CPAD), f32),
        in_specs=[anymem, anymem] + [vmem] * 18,
        out_specs=vmem,
        scratch_shapes=[
            pltpu.VMEM((BLK, 2 * H), f32),
            pltpu.VMEM((8, 4 * H), f32),
            pltpu.SemaphoreType.DMA,
            pltpu.SemaphoreType.DMA,
        ],
        compiler_params=pltpu.CompilerParams(
            vmem_limit_bytes=100 * 1024 * 1024,
        ),
    )(enc, L_ys_w,
      W_s_mono_w, W_s_mono_b.reshape(1, 2 * H), W_h_mono_w, v_mono_w,
      g_mono.reshape(1, 1), r_mono.reshape(1, 1),
      W_s_chunk_w, W_s_chunk_b.reshape(1, 2 * H), W_h_chunk_w, v_chunk_w,
      L_sy_w, L_gy_w, L_gy_b.reshape(1, H), lyy_pad, lyyb_pad,
      L_ss_w, L_gs_w, L_gs_b.reshape(1, 4 * H))
    return out[:, :C]
